# baseline (speedup 1.0000x reference)
import functools

import jax
import jax.numpy as jnp
from jax import lax
from jax.experimental import pallas as pl
from jax.experimental.pallas import tpu as pltpu
from jax.experimental.pallas import tpu_sc as plsc

N = 10000
E = 320000
NC, NS, LANES = 2, 16, 16
NW = NC * NS


def _nbuf(w):
    return 5 if w == 128 else 6


def _chunk(w):
    return 40 if w == 128 else 128
N_PAD = 10240
RPT = N_PAD // NS
DEGW = 16

@functools.cache
def _mesh():
    return plsc.VectorSubcoreMesh(core_axis_name="c", subcore_axis_name="s")


def _fill(ref, val, rows, w):
    def body(r, carry):
        for cc in range(w // LANES):
            ref[r, pl.ds(cc * LANES, LANES)] = jnp.full((LANES,), val, jnp.float32)
        return carry
    lax.fori_loop(0, rows, body, 0)


@functools.cache
def _spmm(w):
    chunk = _chunk(w)
    nbuf = _nbuf(w)
    nrows = E // chunk
    cpw = nrows // NW
    extra = nrows - NW * cpw
    assert cpw % nbuf == 0

    def body(t_hbm, srcp, dstp, out, acc, src_v, dst_v, ex_v, exd_v, *bufsems):
        bufs, sems = bufsems[:nbuf], bufsems[nbuf:]
        c = lax.axis_index("c")
        s = lax.axis_index("s")
        wid = c * NS + s
        base = wid * cpw
        pltpu.sync_copy(srcp.at[pl.ds(base, cpw)], src_v)
        pltpu.sync_copy(dstp.at[pl.ds(base, cpw)], dst_v)
        _fill(bufs[0], 0.0, chunk, w)
        for k in range(RPT // chunk):
            pltpu.sync_copy(bufs[0], acc.at[pl.ds(s * RPT + k * chunk, chunk)])
        plsc.subcore_barrier()

        if chunk == 128:
            lag = nbuf // 2
            gs, ss = sems[:nbuf], sems[nbuf:]

            def _wait_g(jj, b):
                pltpu.make_async_copy(t_hbm.at[src_v.at[jj]], bufs[b], gs[b]).wait()

            def _wait_s(jj, b):
                pltpu.make_async_copy(bufs[b], acc.at[dst_v.at[jj]], ss[b]).wait()

            for b in range(lag):
                pltpu.async_copy(t_hbm.at[src_v.at[b]], bufs[b], gs[b])
            for j in range(lag):
                _wait_g(j, j)
                pltpu.async_copy(bufs[j], acc.at[dst_v.at[j]], ss[j], add=True)
                pltpu.async_copy(t_hbm.at[src_v.at[j + lag]], bufs[j + lag], gs[j + lag])

            def step2(i, carry):
                j0 = nbuf * i + lag
                for t in range(nbuf):
                    j = j0 + t
                    b = (lag + t) % nbuf
                    b2 = (lag + t + lag) % nbuf
                    _wait_g(j, b)
                    pltpu.async_copy(bufs[b], acc.at[dst_v.at[j]], ss[b], add=True)
                    _wait_s(j - lag, b2)
                    pltpu.async_copy(t_hbm.at[src_v.at[j + lag]], bufs[b2], gs[b2])
                return carry

            n_main = (cpw - 2 * lag) // nbuf
            lax.fori_loop(0, n_main, step2, 0)
            for t in range(lag):
                j = cpw - lag + t
                b = (lag + t) % nbuf
                b2 = (t + 2 * lag) % nbuf
                _wait_g(j, b)
                pltpu.async_copy(bufs[b], acc.at[dst_v.at[j]], ss[b], add=True)
                _wait_s(j - lag, b2)
            for t in range(lag):
                j = cpw - lag + t
                b = (lag + t) % nbuf
                _wait_s(j, b)
        else:
            for b in range(nbuf):
                pltpu.async_copy(t_hbm.at[src_v.at[b]], bufs[b], sems[b])

            def step(i, carry):
                j = nbuf * i
                for b in range(nbuf):
                    jj = j + b
                    pltpu.make_async_copy(t_hbm.at[src_v.at[jj]], bufs[b], sems[b]).wait()
                    pltpu.sync_copy(bufs[b], acc.at[dst_v.at[jj]], add=True)
                    pltpu.async_copy(t_hbm.at[src_v.at[jj + nbuf]], bufs[b], sems[b])
                return carry

            lax.fori_loop(0, cpw // nbuf - 1, step, 0)
            for b in range(nbuf):
                jj = cpw - nbuf + b
                pltpu.make_async_copy(t_hbm.at[src_v.at[jj]], bufs[b], sems[b]).wait()
                pltpu.sync_copy(bufs[b], acc.at[dst_v.at[jj]], add=True)
        if extra:
            @pl.when(wid < extra)
            def _():
                pltpu.sync_copy(srcp.at[pl.ds(NW * cpw + wid, 1)], ex_v)
                pltpu.sync_copy(dstp.at[pl.ds(NW * cpw + wid, 1)], exd_v)
                cp = pltpu.async_copy(t_hbm.at[ex_v.at[0]], bufs[0], sems[0])
                cp.wait()
                pltpu.sync_copy(bufs[0], acc.at[exd_v.at[0]], add=True)
        plsc.subcore_barrier()
        if w == 128:
            pltpu.sync_copy(acc.at[pl.ds(s * RPT, RPT)],
                            out.at[c, pl.ds(s * RPT, RPT)])
        else:
            pltpu.sync_copy(acc.at[pl.ds(s * RPT, RPT)],
                            out.at[pl.ds(s * RPT, RPT), pl.ds(c * w, w)])

    out_sh = ((NC, N_PAD, 128) if w == 128 else (N_PAD, 128))
    return pl.kernel(
        body,
        out_type=jax.ShapeDtypeStruct(out_sh, jnp.float32),
        mesh=_mesh(),
        compiler_params=pltpu.CompilerParams(use_tc_tiling_on_sc=False),
        scratch_types=(
            [pltpu.VMEM_SHARED((N_PAD, w), jnp.float32),
             pltpu.VMEM((cpw, chunk), jnp.int32),
             pltpu.VMEM((cpw, chunk), jnp.int32),
             pltpu.VMEM((1, chunk), jnp.int32),
             pltpu.VMEM((1, chunk), jnp.int32)]
            + [pltpu.VMEM((chunk, w), jnp.float32)] * nbuf
            + [pltpu.SemaphoreType.DMA] * (2 * nbuf if chunk == 128 else nbuf)
        ),
    )


def _indeg_kernel():
    chunk = 128
    nrows = E // chunk
    cpw = nrows // NW
    extra = nrows - NW * cpw

    def body(dstp, out, acc, dst_v, ex_v, ones_v, zero_v):
        c = lax.axis_index("c")
        s = lax.axis_index("s")
        wid = c * NS + s
        base = wid * cpw
        pltpu.sync_copy(dstp.at[pl.ds(base, cpw)], dst_v)
        _fill(ones_v, 1.0, chunk, DEGW)
        _fill(zero_v, 0.0, chunk, DEGW)
        for k in range(RPT // chunk):
            pltpu.sync_copy(zero_v, acc.at[pl.ds(s * RPT + k * chunk, chunk)])
        plsc.subcore_barrier()

        def step(j, carry):
            pltpu.sync_copy(ones_v, acc.at[dst_v.at[j]], add=True)
            return carry

        lax.fori_loop(0, cpw, step, 0)
        if extra:
            @pl.when(wid < extra)
            def _():
                pltpu.sync_copy(dstp.at[pl.ds(NW * cpw + wid, 1)], ex_v)
                pltpu.sync_copy(ones_v, acc.at[ex_v.at[0]], add=True)
        plsc.subcore_barrier()
        pltpu.sync_copy(acc.at[pl.ds(s * RPT, RPT)],
                        out.at[pl.ds(s * RPT, RPT), pl.ds(c * DEGW, DEGW)])

    return pl.kernel(
        body,
        out_type=jax.ShapeDtypeStruct((N_PAD, 2 * DEGW), jnp.float32),
        mesh=_mesh(),
        compiler_params=pltpu.CompilerParams(use_tc_tiling_on_sc=False),
        scratch_types=[
            pltpu.VMEM_SHARED((N_PAD, DEGW), jnp.float32),
            pltpu.VMEM((cpw, chunk), jnp.int32),
            pltpu.VMEM((1, chunk), jnp.int32),
            pltpu.VMEM((chunk, DEGW), jnp.float32),
            pltpu.VMEM((chunk, DEGW), jnp.float32),
        ],
    )


RB = 1000
GRID = (N // RB,)


def _rspec(w):
    return pl.BlockSpec((RB, w), lambda i: (i, 0))


def _dspec():
    return pl.BlockSpec((RB, 2 * DEGW), lambda i: (i, 0))


def _pspec(w):
    return pl.BlockSpec((RB, 128), lambda i: (i, 0))


def _pspec3(w):
    return pl.BlockSpec((NC, RB, w), lambda i: (0, i, 0))


def _psum(p, w):
    return p[:, :w] + p[:, w:2 * w]


def _wspec(shape):
    return pl.BlockSpec(shape, lambda i: (0,) * len(shape))


def _scales(pdeg):
    indeg = pdeg[:, 0:1] + pdeg[:, DEGW:DEGW + 1]
    dinv = lax.rsqrt(indeg + 2.0)
    sc1 = dinv / (indeg + 1.0)
    return dinv, sc1


def _elu(v):
    return jnp.where(v > 0, v, jnp.exp(jnp.minimum(v, 0.0)) - 1.0)


def _sig(v):
    return 1.0 / (1.0 + jnp.exp(-v))


def _gate(z, oc):
    return _elu(z[:, :oc]) * _sig(z[:, oc:])


def _init_body(x_ref, w_ref, o_ref):
    o_ref[...] = jnp.dot(x_ref[...], w_ref[...], preferred_element_type=jnp.float32)


def _pre0_body(pdeg_ref, u_ref, o_ref):
    dinv, _ = _scales(pdeg_ref[...])
    o_ref[...] = dinv * u_ref[...]


def _stage0_body(pdeg_ref, p_ref, t_ref, out0_ref, t1_ref):
    dinv, sc1 = _scales(pdeg_ref[...])
    z = sc1 * (_psum(p_ref[...], 64) + 2.0 * t_ref[...])
    g = _gate(z, 32)
    out0_ref[...] = g
    t1_ref[...] = dinv * g


def _stage_mid(pdeg_ref, p_ref, t_ref, w_ref, o_refs, oc, skip_ref=None):
    dinv, sc1 = _scales(pdeg_ref[...])
    p = p_ref[...]
    psum = (p[0] + p[1]) if p.ndim == 3 else _psum(p, t_ref.shape[-1])
    y = sc1 * (psum + 2.0 * t_ref[...])
    h = jnp.dot(y, w_ref[...], preferred_element_type=jnp.float32)
    g = _gate(h, oc)
    if skip_ref is not None:
        g = g + skip_ref[...]
    if len(o_refs) == 2:
        o_refs[0][...] = g
        o_refs[1][...] = dinv * g
    else:
        o_refs[0][...] = dinv * g


def _stage1_body(pdeg_ref, p_ref, t_ref, w_ref, out_ref, tn_ref):
    _stage_mid(pdeg_ref, p_ref, t_ref, w_ref, (out_ref, tn_ref), 64)


def _stage2_body(pdeg_ref, p_ref, t_ref, w_ref, tn_ref):
    _stage_mid(pdeg_ref, p_ref, t_ref, w_ref, (tn_ref,), 128)


def _stage3_body(pdeg_ref, p_ref, t_ref, w_ref, skip_ref, tn_ref):
    _stage_mid(pdeg_ref, p_ref, t_ref, w_ref, (tn_ref,), 64, skip_ref)


def _stage4_body(pdeg_ref, p_ref, t_ref, w_ref, skip_ref, w5_ref, t5_ref):
    dinv, sc1 = _scales(pdeg_ref[...])
    y = sc1 * (_psum(p_ref[...], 64) + 2.0 * t_ref[...])
    h = jnp.dot(y, w_ref[...], preferred_element_type=jnp.float32)
    g = _gate(h, 32) + skip_ref[...]
    u5 = jnp.dot(g, w5_ref[...], preferred_element_type=jnp.float32)
    t5_ref[...] = dinv * u5


def _stage5_body(pdeg_ref, p_ref, t_ref, o_ref):
    _, sc1 = _scales(pdeg_ref[...])
    z = sc1 * (_psum(p_ref[...], DEGW) + 2.0 * t_ref[...])
    o_ref[...] = _sig(z[:, :3])


def _f32(shape):
    return jax.ShapeDtypeStruct(shape, jnp.float32)


def kernel(x, edge_index, W0a, W0b, W1a, W1b, W2a, W2b, W3a, W3b, W4a, W4b, W5):
    src = edge_index[0].astype(jnp.int32)
    dst = edge_index[1].astype(jnp.int32)
    srcp = src.reshape(-1, 128)
    dstp = dst.reshape(-1, 128)
    srcp40 = src.reshape(-1, 40)
    dstp40 = dst.reshape(-1, 40)

    W0 = jnp.concatenate([W0a, W0b], axis=1)
    W1 = jnp.concatenate([W1a, W1b], axis=1)
    W2 = jnp.concatenate([W2a, W2b], axis=1)
    W3 = jnp.concatenate([W3a, W3b], axis=1)
    W4 = jnp.concatenate([W4a, W4b], axis=1)
    W5p = jnp.pad(W5, ((0, 0), (0, DEGW - 3)))

    pdeg = _indeg_kernel()(dstp)

    u0 = pl.pallas_call(
        _init_body, grid=GRID,
        in_specs=[_rspec(128), _wspec((128, 64))],
        out_specs=_rspec(64), out_shape=_f32((N, 64)),
    )(x, W0)

    t0 = pl.pallas_call(
        _pre0_body, grid=GRID,
        in_specs=[_dspec(), _rspec(64)],
        out_specs=_rspec(64), out_shape=_f32((N, 64)),
    )(pdeg, u0)

    p0 = _spmm(64)(t0, srcp, dstp)
    out0, t1 = pl.pallas_call(
        _stage0_body, grid=GRID,
        in_specs=[_dspec(), _pspec(64), _rspec(64)],
        out_specs=(_rspec(32), _rspec(32)),
        out_shape=(_f32((N, 32)), _f32((N, 32))),
    )(pdeg, p0, t0)

    p1 = _spmm(32)(t1, srcp, dstp)
    out1, t2 = pl.pallas_call(
        _stage1_body, grid=GRID,
        in_specs=[_dspec(), _pspec(32), _rspec(32), _wspec((32, 128))],
        out_specs=(_rspec(64), _rspec(64)),
        out_shape=(_f32((N, 64)), _f32((N, 64))),
    )(pdeg, p1, t1, W1)

    p2 = _spmm(64)(t2, srcp, dstp)
    (t3,) = pl.pallas_call(
        _stage2_body, grid=GRID,
        in_specs=[_dspec(), _pspec(64), _rspec(64), _wspec((64, 256))],
        out_specs=(_rspec(128),),
        out_shape=(_f32((N, 128)),),
    )(pdeg, p2, t2, W2)

    p3 = _spmm(128)(t3, srcp40, dstp40)
    (t4,) = pl.pallas_call(
        _stage3_body, grid=GRID,
        in_specs=[_dspec(), _pspec3(128), _rspec(128), _wspec((128, 128)),
                  _rspec(64)],
        out_specs=(_rspec(64),),
        out_shape=(_f32((N, 64)),),
    )(pdeg, p3, t3, W3, out1)

    p4 = _spmm(64)(t4, srcp, dstp)
    (t5,) = pl.pallas_call(
        _stage4_body, grid=GRID,
        in_specs=[_dspec(), _pspec(64), _rspec(64), _wspec((64, 64)),
                  _rspec(32), _wspec((32, DEGW))],
        out_specs=(_rspec(DEGW),),
        out_shape=(_f32((N, DEGW)),),
    )(pdeg, p4, t4, W4, out0, W5p)

    p5 = _spmm(DEGW)(t5, srcp, dstp)
    return pl.pallas_call(
        _stage5_body, grid=GRID,
        in_specs=[_dspec(), _pspec(DEGW), _rspec(DEGW)],
        out_specs=_rspec(3), out_shape=_f32((N, 3)),
    )(pdeg, p5, t5)


# speedup vs baseline: 1.0698x; 1.0698x over previous
import functools

import jax
import jax.numpy as jnp
from jax import lax
from jax.experimental import pallas as pl
from jax.experimental.pallas import tpu as pltpu
from jax.experimental.pallas import tpu_sc as plsc

N = 10000
E = 320000
NC, NS, LANES = 2, 16, 16
NW = NC * NS


def _nbuf(w):
    return 5 if w == 128 else 6


def _chunk(w):
    return 40 if w == 128 else 128
N_PAD = 10240
RPT = N_PAD // NS
DEGW = 16

@functools.cache
def _mesh():
    return plsc.VectorSubcoreMesh(core_axis_name="c", subcore_axis_name="s")


def _fill(ref, val, rows, w):
    def body(r, carry):
        for cc in range(w // LANES):
            ref[r, pl.ds(cc * LANES, LANES)] = jnp.full((LANES,), val, jnp.float32)
        return carry
    lax.fori_loop(0, rows, body, 0)


@functools.cache
def _spmm(w):
    chunk = _chunk(w)
    nbuf = _nbuf(w)
    nrows = E // chunk
    cpw = nrows // NW
    extra = nrows - NW * cpw
    assert cpw % nbuf == 0

    def body(t_hbm, srcp, dstp, out, acc, src_v, dst_v, ex_v, exd_v, *bufsems):
        bufs, sems = bufsems[:nbuf], bufsems[nbuf:]
        c = lax.axis_index("c")
        s = lax.axis_index("s")
        wid = c * NS + s
        base = wid * cpw
        pltpu.sync_copy(srcp.at[pl.ds(base, cpw)], src_v)
        pltpu.sync_copy(dstp.at[pl.ds(base, cpw)], dst_v)
        _fill(bufs[0], 0.0, chunk, w)
        for k in range(RPT // chunk):
            pltpu.sync_copy(bufs[0], acc.at[pl.ds(s * RPT + k * chunk, chunk)])
        plsc.subcore_barrier()

        for b in range(nbuf):
            pltpu.async_copy(t_hbm.at[src_v.at[b]], bufs[b], sems[b])

        def step(i, carry):
            j = nbuf * i
            for b in range(nbuf):
                jj = j + b
                pltpu.make_async_copy(t_hbm.at[src_v.at[jj]], bufs[b], sems[b]).wait()
                pltpu.sync_copy(bufs[b], acc.at[dst_v.at[jj]], add=True)
                pltpu.async_copy(t_hbm.at[src_v.at[jj + nbuf]], bufs[b], sems[b])
            return carry

        lax.fori_loop(0, cpw // nbuf - 1, step, 0)
        for b in range(nbuf):
            jj = cpw - nbuf + b
            pltpu.make_async_copy(t_hbm.at[src_v.at[jj]], bufs[b], sems[b]).wait()
            pltpu.sync_copy(bufs[b], acc.at[dst_v.at[jj]], add=True)
        if extra:
            @pl.when(wid < extra)
            def _():
                pltpu.sync_copy(srcp.at[pl.ds(NW * cpw + wid, 1)], ex_v)
                pltpu.sync_copy(dstp.at[pl.ds(NW * cpw + wid, 1)], exd_v)
                cp = pltpu.async_copy(t_hbm.at[ex_v.at[0]], bufs[0], sems[0])
                cp.wait()
                pltpu.sync_copy(bufs[0], acc.at[exd_v.at[0]], add=True)
        plsc.subcore_barrier()
        if w == 128:
            pltpu.sync_copy(acc.at[pl.ds(s * RPT, RPT)],
                            out.at[c, pl.ds(s * RPT, RPT)])
        else:
            pltpu.sync_copy(acc.at[pl.ds(s * RPT, RPT)],
                            out.at[pl.ds(s * RPT, RPT), pl.ds(c * w, w)])

    out_sh = ((NC, N_PAD, 128) if w == 128 else (N_PAD, 128))
    return pl.kernel(
        body,
        out_type=jax.ShapeDtypeStruct(out_sh, jnp.float32),
        mesh=_mesh(),
        compiler_params=pltpu.CompilerParams(use_tc_tiling_on_sc=False),
        scratch_types=(
            [pltpu.VMEM_SHARED((N_PAD, w), jnp.float32),
             pltpu.VMEM((cpw, chunk), jnp.int32),
             pltpu.VMEM((cpw, chunk), jnp.int32),
             pltpu.VMEM((1, chunk), jnp.int32),
             pltpu.VMEM((1, chunk), jnp.int32)]
            + [pltpu.VMEM((chunk, w), jnp.float32)] * nbuf
            + [pltpu.SemaphoreType.DMA] * nbuf
        ),
    )


def _indeg_kernel():
    chunk = 128
    nrows = E // chunk
    cpw = nrows // NW
    extra = nrows - NW * cpw

    def body(dstp, out, acc, dst_v, ex_v, ones_v, zero_v):
        c = lax.axis_index("c")
        s = lax.axis_index("s")
        wid = c * NS + s
        base = wid * cpw
        pltpu.sync_copy(dstp.at[pl.ds(base, cpw)], dst_v)
        _fill(ones_v, 1.0, chunk, DEGW)
        _fill(zero_v, 0.0, chunk, DEGW)
        for k in range(RPT // chunk):
            pltpu.sync_copy(zero_v, acc.at[pl.ds(s * RPT + k * chunk, chunk)])
        plsc.subcore_barrier()

        def step(j, carry):
            pltpu.sync_copy(ones_v, acc.at[dst_v.at[j]], add=True)
            return carry

        lax.fori_loop(0, cpw, step, 0)
        if extra:
            @pl.when(wid < extra)
            def _():
                pltpu.sync_copy(dstp.at[pl.ds(NW * cpw + wid, 1)], ex_v)
                pltpu.sync_copy(ones_v, acc.at[ex_v.at[0]], add=True)
        plsc.subcore_barrier()
        pltpu.sync_copy(acc.at[pl.ds(s * RPT, RPT)],
                        out.at[pl.ds(s * RPT, RPT), pl.ds(c * DEGW, DEGW)])

    return pl.kernel(
        body,
        out_type=jax.ShapeDtypeStruct((N_PAD, 2 * DEGW), jnp.float32),
        mesh=_mesh(),
        compiler_params=pltpu.CompilerParams(use_tc_tiling_on_sc=False),
        scratch_types=[
            pltpu.VMEM_SHARED((N_PAD, DEGW), jnp.float32),
            pltpu.VMEM((cpw, chunk), jnp.int32),
            pltpu.VMEM((1, chunk), jnp.int32),
            pltpu.VMEM((chunk, DEGW), jnp.float32),
            pltpu.VMEM((chunk, DEGW), jnp.float32),
        ],
    )


RB = 1000
GRID = (N // RB,)


def _rspec(w):
    return pl.BlockSpec((RB, w), lambda i: (i, 0))


def _dspec():
    return pl.BlockSpec((RB, 2 * DEGW), lambda i: (i, 0))


def _pspec(w):
    return pl.BlockSpec((RB, 128), lambda i: (i, 0))


def _pspec3(w):
    return pl.BlockSpec((NC, RB, w), lambda i: (0, i, 0))


def _psum(p, w):
    return p[:, :w] + p[:, w:2 * w]


def _wspec(shape):
    return pl.BlockSpec(shape, lambda i: (0,) * len(shape))


def _scales(pdeg):
    indeg = pdeg[:, 0:1] + pdeg[:, DEGW:DEGW + 1]
    dinv = lax.rsqrt(indeg + 2.0)
    sc1 = dinv / (indeg + 1.0)
    return dinv, sc1


def _elu(v):
    return jnp.where(v > 0, v, jnp.exp(jnp.minimum(v, 0.0)) - 1.0)


def _sig(v):
    return 1.0 / (1.0 + jnp.exp(-v))


def _gate(z, oc):
    return _elu(z[:, :oc]) * _sig(z[:, oc:])


def _init_body(x_ref, w_ref, o_ref):
    o_ref[...] = jnp.dot(x_ref[...], w_ref[...], preferred_element_type=jnp.float32)


def _pre0_body(pdeg_ref, u_ref, o_ref):
    dinv, _ = _scales(pdeg_ref[...])
    o_ref[...] = dinv * u_ref[...]


def _stage0_body(pdeg_ref, p_ref, t_ref, out0_ref, t1_ref):
    dinv, sc1 = _scales(pdeg_ref[...])
    z = sc1 * (_psum(p_ref[...], 64) + 2.0 * t_ref[...])
    g = _gate(z, 32)
    out0_ref[...] = g
    t1_ref[...] = dinv * g


def _stage_mid(pdeg_ref, p_ref, t_ref, w_ref, o_refs, oc, skip_ref=None):
    dinv, sc1 = _scales(pdeg_ref[...])
    p = p_ref[...]
    psum = (p[0] + p[1]) if p.ndim == 3 else _psum(p, t_ref.shape[-1])
    y = sc1 * (psum + 2.0 * t_ref[...])
    h = jnp.dot(y, w_ref[...], preferred_element_type=jnp.float32)
    g = _gate(h, oc)
    if skip_ref is not None:
        g = g + skip_ref[...]
    if len(o_refs) == 2:
        o_refs[0][...] = g
        o_refs[1][...] = dinv * g
    else:
        o_refs[0][...] = dinv * g


def _stage1_body(pdeg_ref, p_ref, t_ref, w_ref, out_ref, tn_ref):
    _stage_mid(pdeg_ref, p_ref, t_ref, w_ref, (out_ref, tn_ref), 64)


def _stage2_body(pdeg_ref, p_ref, t_ref, w_ref, tn_ref):
    _stage_mid(pdeg_ref, p_ref, t_ref, w_ref, (tn_ref,), 128)


def _stage3_body(pdeg_ref, p_ref, t_ref, w_ref, skip_ref, tn_ref):
    _stage_mid(pdeg_ref, p_ref, t_ref, w_ref, (tn_ref,), 64, skip_ref)


def _stage4_body(pdeg_ref, p_ref, t_ref, w_ref, skip_ref, w5_ref, t5_ref):
    dinv, sc1 = _scales(pdeg_ref[...])
    y = sc1 * (_psum(p_ref[...], 64) + 2.0 * t_ref[...])
    h = jnp.dot(y, w_ref[...], preferred_element_type=jnp.float32)
    g = _gate(h, 32) + skip_ref[...]
    u5 = jnp.dot(g, w5_ref[...], preferred_element_type=jnp.float32)
    t5_ref[...] = dinv * u5


def _stage5_body(pdeg_ref, p_ref, t_ref, o_ref):
    _, sc1 = _scales(pdeg_ref[...])
    z = sc1 * (_psum(p_ref[...], DEGW) + 2.0 * t_ref[...])
    o_ref[...] = _sig(z[:, :3])


def _f32(shape):
    return jax.ShapeDtypeStruct(shape, jnp.float32)


def kernel(x, edge_index, W0a, W0b, W1a, W1b, W2a, W2b, W3a, W3b, W4a, W4b, W5):
    src = edge_index[0].astype(jnp.int32)
    dst = edge_index[1].astype(jnp.int32)
    srcp = src.reshape(-1, 128)
    dstp = dst.reshape(-1, 128)
    srcp40 = src.reshape(-1, 40)
    dstp40 = dst.reshape(-1, 40)

    W0 = jnp.concatenate([W0a, W0b], axis=1)
    W1 = jnp.concatenate([W1a, W1b], axis=1)
    W2 = jnp.concatenate([W2a, W2b], axis=1)
    W3 = jnp.concatenate([W3a, W3b], axis=1)
    W4 = jnp.concatenate([W4a, W4b], axis=1)
    W5p = jnp.pad(W5, ((0, 0), (0, DEGW - 3)))

    pdeg = _indeg_kernel()(dstp)

    u0 = pl.pallas_call(
        _init_body, grid=GRID,
        in_specs=[_rspec(128), _wspec((128, 64))],
        out_specs=_rspec(64), out_shape=_f32((N, 64)),
    )(x, W0)

    t0 = pl.pallas_call(
        _pre0_body, grid=GRID,
        in_specs=[_dspec(), _rspec(64)],
        out_specs=_rspec(64), out_shape=_f32((N, 64)),
    )(pdeg, u0)

    p0 = _spmm(64)(t0, srcp, dstp)
    out0, t1 = pl.pallas_call(
        _stage0_body, grid=GRID,
        in_specs=[_dspec(), _pspec(64), _rspec(64)],
        out_specs=(_rspec(32), _rspec(32)),
        out_shape=(_f32((N, 32)), _f32((N, 32))),
    )(pdeg, p0, t0)

    p1 = _spmm(32)(t1, srcp, dstp)
    out1, t2 = pl.pallas_call(
        _stage1_body, grid=GRID,
        in_specs=[_dspec(), _pspec(32), _rspec(32), _wspec((32, 128))],
        out_specs=(_rspec(64), _rspec(64)),
        out_shape=(_f32((N, 64)), _f32((N, 64))),
    )(pdeg, p1, t1, W1)

    p2 = _spmm(64)(t2, srcp, dstp)
    (t3,) = pl.pallas_call(
        _stage2_body, grid=GRID,
        in_specs=[_dspec(), _pspec(64), _rspec(64), _wspec((64, 256))],
        out_specs=(_rspec(128),),
        out_shape=(_f32((N, 128)),),
    )(pdeg, p2, t2, W2)

    p3 = _spmm(128)(t3, srcp40, dstp40)
    (t4,) = pl.pallas_call(
        _stage3_body, grid=GRID,
        in_specs=[_dspec(), _pspec3(128), _rspec(128), _wspec((128, 128)),
                  _rspec(64)],
        out_specs=(_rspec(64),),
        out_shape=(_f32((N, 64)),),
    )(pdeg, p3, t3, W3, out1)

    p4 = _spmm(64)(t4, srcp, dstp)
    (t5,) = pl.pallas_call(
        _stage4_body, grid=GRID,
        in_specs=[_dspec(), _pspec(64), _rspec(64), _wspec((64, 64)),
                  _rspec(32), _wspec((32, DEGW))],
        out_specs=(_rspec(DEGW),),
        out_shape=(_f32((N, DEGW)),),
    )(pdeg, p4, t4, W4, out0, W5p)

    p5 = _spmm(DEGW)(t5, srcp, dstp)
    return pl.pallas_call(
        _stage5_body, grid=GRID,
        in_specs=[_dspec(), _pspec(DEGW), _rspec(DEGW)],
        out_specs=_rspec(3), out_shape=_f32((N, 3)),
    )(pdeg, p5, t5)


# speedup vs baseline: 1.0786x; 1.0082x over previous
import functools

import jax
import jax.numpy as jnp
from jax import lax
from jax.experimental import pallas as pl
from jax.experimental.pallas import tpu as pltpu
from jax.experimental.pallas import tpu_sc as plsc

N = 10000
E = 320000
NC, NS, LANES = 2, 16, 16
NW = NC * NS


def _nbuf(w):
    if w == 128:
        return 5
    return 13 if w <= 32 else 6


def _chunk(w):
    return 40 if w == 128 else 128
N_PAD = 10240
RPT = N_PAD // NS
DEGW = 16

@functools.cache
def _mesh():
    return plsc.VectorSubcoreMesh(core_axis_name="c", subcore_axis_name="s")


def _fill(ref, val, rows, w):
    def body(r, carry):
        for cc in range(w // LANES):
            ref[r, pl.ds(cc * LANES, LANES)] = jnp.full((LANES,), val, jnp.float32)
        return carry
    lax.fori_loop(0, rows, body, 0)


@functools.cache
def _spmm(w):
    chunk = _chunk(w)
    nbuf = _nbuf(w)
    nrows = E // chunk
    cpw = nrows // NW
    extra = nrows - NW * cpw
    assert cpw % nbuf == 0

    def body(t_hbm, srcp, dstp, out, acc, src_v, dst_v, ex_v, exd_v, *bufsems):
        bufs, sems = bufsems[:nbuf], bufsems[nbuf:]
        c = lax.axis_index("c")
        s = lax.axis_index("s")
        wid = c * NS + s
        base = wid * cpw
        pltpu.sync_copy(srcp.at[pl.ds(base, cpw)], src_v)
        pltpu.sync_copy(dstp.at[pl.ds(base, cpw)], dst_v)
        _fill(bufs[0], 0.0, chunk, w)
        for k in range(RPT // chunk):
            pltpu.sync_copy(bufs[0], acc.at[pl.ds(s * RPT + k * chunk, chunk)])
        plsc.subcore_barrier()

        for b in range(nbuf):
            pltpu.async_copy(t_hbm.at[src_v.at[b]], bufs[b], sems[b])

        def step(i, carry):
            j = nbuf * i
            for b in range(nbuf):
                jj = j + b
                pltpu.make_async_copy(t_hbm.at[src_v.at[jj]], bufs[b], sems[b]).wait()
                pltpu.sync_copy(bufs[b], acc.at[dst_v.at[jj]], add=True)
                pltpu.async_copy(t_hbm.at[src_v.at[jj + nbuf]], bufs[b], sems[b])
            return carry

        lax.fori_loop(0, cpw // nbuf - 1, step, 0)
        for b in range(nbuf):
            jj = cpw - nbuf + b
            pltpu.make_async_copy(t_hbm.at[src_v.at[jj]], bufs[b], sems[b]).wait()
            pltpu.sync_copy(bufs[b], acc.at[dst_v.at[jj]], add=True)
        if extra:
            @pl.when(wid < extra)
            def _():
                pltpu.sync_copy(srcp.at[pl.ds(NW * cpw + wid, 1)], ex_v)
                pltpu.sync_copy(dstp.at[pl.ds(NW * cpw + wid, 1)], exd_v)
                cp = pltpu.async_copy(t_hbm.at[ex_v.at[0]], bufs[0], sems[0])
                cp.wait()
                pltpu.sync_copy(bufs[0], acc.at[exd_v.at[0]], add=True)
        plsc.subcore_barrier()
        if w == 128:
            pltpu.sync_copy(acc.at[pl.ds(s * RPT, RPT)],
                            out.at[c, pl.ds(s * RPT, RPT)])
        else:
            pltpu.sync_copy(acc.at[pl.ds(s * RPT, RPT)],
                            out.at[pl.ds(s * RPT, RPT), pl.ds(c * w, w)])

    out_sh = ((NC, N_PAD, 128) if w == 128 else (N_PAD, 128))
    return pl.kernel(
        body,
        out_type=jax.ShapeDtypeStruct(out_sh, jnp.float32),
        mesh=_mesh(),
        compiler_params=pltpu.CompilerParams(use_tc_tiling_on_sc=False),
        scratch_types=(
            [pltpu.VMEM_SHARED((N_PAD, w), jnp.float32),
             pltpu.VMEM((cpw, chunk), jnp.int32),
             pltpu.VMEM((cpw, chunk), jnp.int32),
             pltpu.VMEM((1, chunk), jnp.int32),
             pltpu.VMEM((1, chunk), jnp.int32)]
            + [pltpu.VMEM((chunk, w), jnp.float32)] * nbuf
            + [pltpu.SemaphoreType.DMA] * nbuf
        ),
    )


def _indeg_kernel():
    chunk = 128
    nrows = E // chunk
    cpw = nrows // NW
    extra = nrows - NW * cpw

    def body(dstp, out, acc, dst_v, ex_v, ones_v, zero_v, sem):
        c = lax.axis_index("c")
        s = lax.axis_index("s")
        wid = c * NS + s
        base = wid * cpw
        pltpu.sync_copy(dstp.at[pl.ds(base, cpw)], dst_v)
        _fill(ones_v, 1.0, chunk, DEGW)
        _fill(zero_v, 0.0, chunk, DEGW)
        for k in range(RPT // chunk):
            pltpu.sync_copy(zero_v, acc.at[pl.ds(s * RPT + k * chunk, chunk)])
        plsc.subcore_barrier()

        def step(j, carry):
            pltpu.async_copy(ones_v, acc.at[dst_v.at[j]], sem, add=True)
            return carry

        lax.fori_loop(0, cpw, step, 0)

        def drain(j, carry):
            pltpu.make_async_copy(ones_v, acc.at[dst_v.at[j]], sem).wait()
            return carry

        lax.fori_loop(0, cpw, drain, 0)
        if extra:
            @pl.when(wid < extra)
            def _():
                pltpu.sync_copy(dstp.at[pl.ds(NW * cpw + wid, 1)], ex_v)
                pltpu.sync_copy(ones_v, acc.at[ex_v.at[0]], add=True)
        plsc.subcore_barrier()
        pltpu.sync_copy(acc.at[pl.ds(s * RPT, RPT)],
                        out.at[pl.ds(s * RPT, RPT), pl.ds(c * DEGW, DEGW)])

    return pl.kernel(
        body,
        out_type=jax.ShapeDtypeStruct((N_PAD, 2 * DEGW), jnp.float32),
        mesh=_mesh(),
        compiler_params=pltpu.CompilerParams(use_tc_tiling_on_sc=False),
        scratch_types=[
            pltpu.VMEM_SHARED((N_PAD, DEGW), jnp.float32),
            pltpu.VMEM((cpw, chunk), jnp.int32),
            pltpu.VMEM((1, chunk), jnp.int32),
            pltpu.VMEM((chunk, DEGW), jnp.float32),
            pltpu.VMEM((chunk, DEGW), jnp.float32),
            pltpu.SemaphoreType.DMA,
        ],
    )


RB = 1000
GRID = (N // RB,)


def _rspec(w):
    return pl.BlockSpec((RB, w), lambda i: (i, 0))


def _dspec():
    return pl.BlockSpec((RB, 2 * DEGW), lambda i: (i, 0))


def _pspec(w):
    return pl.BlockSpec((RB, 128), lambda i: (i, 0))


def _pspec3(w):
    return pl.BlockSpec((NC, RB, w), lambda i: (0, i, 0))


def _psum(p, w):
    return p[:, :w] + p[:, w:2 * w]


def _wspec(shape):
    return pl.BlockSpec(shape, lambda i: (0,) * len(shape))


def _scales(pdeg):
    indeg = pdeg[:, 0:1] + pdeg[:, DEGW:DEGW + 1]
    dinv = lax.rsqrt(indeg + 2.0)
    sc1 = dinv / (indeg + 1.0)
    return dinv, sc1


def _elu(v):
    return jnp.where(v > 0, v, jnp.exp(jnp.minimum(v, 0.0)) - 1.0)


def _sig(v):
    return 1.0 / (1.0 + jnp.exp(-v))


def _gate(z, oc):
    return _elu(z[:, :oc]) * _sig(z[:, oc:])


def _init_body(x_ref, w_ref, o_ref):
    o_ref[...] = jnp.dot(x_ref[...], w_ref[...], preferred_element_type=jnp.float32)


def _pre0_body(pdeg_ref, u_ref, o_ref):
    dinv, _ = _scales(pdeg_ref[...])
    o_ref[...] = dinv * u_ref[...]


def _stage0_body(pdeg_ref, p_ref, t_ref, out0_ref, t1_ref):
    dinv, sc1 = _scales(pdeg_ref[...])
    z = sc1 * (_psum(p_ref[...], 64) + 2.0 * t_ref[...])
    g = _gate(z, 32)
    out0_ref[...] = g
    t1_ref[...] = dinv * g


def _stage_mid(pdeg_ref, p_ref, t_ref, w_ref, o_refs, oc, skip_ref=None):
    dinv, sc1 = _scales(pdeg_ref[...])
    p = p_ref[...]
    psum = (p[0] + p[1]) if p.ndim == 3 else _psum(p, t_ref.shape[-1])
    y = sc1 * (psum + 2.0 * t_ref[...])
    h = jnp.dot(y, w_ref[...], preferred_element_type=jnp.float32)
    g = _gate(h, oc)
    if skip_ref is not None:
        g = g + skip_ref[...]
    if len(o_refs) == 2:
        o_refs[0][...] = g
        o_refs[1][...] = dinv * g
    else:
        o_refs[0][...] = dinv * g


def _stage1_body(pdeg_ref, p_ref, t_ref, w_ref, out_ref, tn_ref):
    _stage_mid(pdeg_ref, p_ref, t_ref, w_ref, (out_ref, tn_ref), 64)


def _stage2_body(pdeg_ref, p_ref, t_ref, w_ref, tn_ref):
    _stage_mid(pdeg_ref, p_ref, t_ref, w_ref, (tn_ref,), 128)


def _stage3_body(pdeg_ref, p_ref, t_ref, w_ref, skip_ref, tn_ref):
    _stage_mid(pdeg_ref, p_ref, t_ref, w_ref, (tn_ref,), 64, skip_ref)


def _stage4_body(pdeg_ref, p_ref, t_ref, w_ref, skip_ref, w5_ref, t5_ref):
    dinv, sc1 = _scales(pdeg_ref[...])
    y = sc1 * (_psum(p_ref[...], 64) + 2.0 * t_ref[...])
    h = jnp.dot(y, w_ref[...], preferred_element_type=jnp.float32)
    g = _gate(h, 32) + skip_ref[...]
    u5 = jnp.dot(g, w5_ref[...], preferred_element_type=jnp.float32)
    t5_ref[...] = dinv * u5


def _stage5_body(pdeg_ref, p_ref, t_ref, o_ref):
    _, sc1 = _scales(pdeg_ref[...])
    z = sc1 * (_psum(p_ref[...], DEGW) + 2.0 * t_ref[...])
    o_ref[...] = _sig(z[:, :3])


def _f32(shape):
    return jax.ShapeDtypeStruct(shape, jnp.float32)


def kernel(x, edge_index, W0a, W0b, W1a, W1b, W2a, W2b, W3a, W3b, W4a, W4b, W5):
    src = edge_index[0].astype(jnp.int32)
    dst = edge_index[1].astype(jnp.int32)
    srcp = src.reshape(-1, 128)
    dstp = dst.reshape(-1, 128)
    srcp40 = src.reshape(-1, 40)
    dstp40 = dst.reshape(-1, 40)

    W0 = jnp.concatenate([W0a, W0b], axis=1)
    W1 = jnp.concatenate([W1a, W1b], axis=1)
    W2 = jnp.concatenate([W2a, W2b], axis=1)
    W3 = jnp.concatenate([W3a, W3b], axis=1)
    W4 = jnp.concatenate([W4a, W4b], axis=1)
    W5p = jnp.pad(W5, ((0, 0), (0, DEGW - 3)))

    pdeg = _indeg_kernel()(dstp)

    u0 = pl.pallas_call(
        _init_body, grid=GRID,
        in_specs=[_rspec(128), _wspec((128, 64))],
        out_specs=_rspec(64), out_shape=_f32((N, 64)),
    )(x, W0)

    t0 = pl.pallas_call(
        _pre0_body, grid=GRID,
        in_specs=[_dspec(), _rspec(64)],
        out_specs=_rspec(64), out_shape=_f32((N, 64)),
    )(pdeg, u0)

    p0 = _spmm(64)(t0, srcp, dstp)
    out0, t1 = pl.pallas_call(
        _stage0_body, grid=GRID,
        in_specs=[_dspec(), _pspec(64), _rspec(64)],
        out_specs=(_rspec(32), _rspec(32)),
        out_shape=(_f32((N, 32)), _f32((N, 32))),
    )(pdeg, p0, t0)

    p1 = _spmm(32)(t1, srcp, dstp)
    out1, t2 = pl.pallas_call(
        _stage1_body, grid=GRID,
        in_specs=[_dspec(), _pspec(32), _rspec(32), _wspec((32, 128))],
        out_specs=(_rspec(64), _rspec(64)),
        out_shape=(_f32((N, 64)), _f32((N, 64))),
    )(pdeg, p1, t1, W1)

    p2 = _spmm(64)(t2, srcp, dstp)
    (t3,) = pl.pallas_call(
        _stage2_body, grid=GRID,
        in_specs=[_dspec(), _pspec(64), _rspec(64), _wspec((64, 256))],
        out_specs=(_rspec(128),),
        out_shape=(_f32((N, 128)),),
    )(pdeg, p2, t2, W2)

    p3 = _spmm(128)(t3, srcp40, dstp40)
    (t4,) = pl.pallas_call(
        _stage3_body, grid=GRID,
        in_specs=[_dspec(), _pspec3(128), _rspec(128), _wspec((128, 128)),
                  _rspec(64)],
        out_specs=(_rspec(64),),
        out_shape=(_f32((N, 64)),),
    )(pdeg, p3, t3, W3, out1)

    p4 = _spmm(64)(t4, srcp, dstp)
    (t5,) = pl.pallas_call(
        _stage4_body, grid=GRID,
        in_specs=[_dspec(), _pspec(64), _rspec(64), _wspec((64, 64)),
                  _rspec(32), _wspec((32, DEGW))],
        out_specs=(_rspec(DEGW),),
        out_shape=(_f32((N, DEGW)),),
    )(pdeg, p4, t4, W4, out0, W5p)

    p5 = _spmm(DEGW)(t5, srcp, dstp)
    return pl.pallas_call(
        _stage5_body, grid=GRID,
        in_specs=[_dspec(), _pspec(DEGW), _rspec(DEGW)],
        out_specs=_rspec(3), out_shape=_f32((N, 3)),
    )(pdeg, p5, t5)


# speedup vs baseline: 1.1089x; 1.0280x over previous
import functools

import jax
import jax.numpy as jnp
from jax import lax
from jax.experimental import pallas as pl
from jax.experimental.pallas import tpu as pltpu
from jax.experimental.pallas import tpu_sc as plsc

N = 10000
E = 320000
NC, NS, LANES = 2, 16, 16
NW = NC * NS


def _nbuf(w):
    if w == 128:
        return 5
    return 13 if w <= 32 else 6


def _chunk(w):
    return 40 if w == 128 else 128
N_PAD = 10240
RPT = N_PAD // NS
DEGW = 16

@functools.cache
def _mesh():
    return plsc.VectorSubcoreMesh(core_axis_name="c", subcore_axis_name="s")


def _fill(ref, val, rows, w):
    def body(r, carry):
        for cc in range(w // LANES):
            ref[r, pl.ds(cc * LANES, LANES)] = jnp.full((LANES,), val, jnp.float32)
        return carry
    lax.fori_loop(0, rows, body, 0)


@functools.cache
def _spmm(w):
    chunk = _chunk(w)
    nbuf = _nbuf(w)
    nrows = E // chunk
    cpw = nrows // NW
    extra = nrows - NW * cpw
    assert cpw % nbuf == 0

    def body(t_hbm, srcp, dstp, out, acc, src_v, dst_v, ex_v, exd_v, *bufsems):
        bufs, sems = bufsems[:nbuf], bufsems[nbuf:]
        c = lax.axis_index("c")
        s = lax.axis_index("s")
        wid = c * NS + s
        base = wid * cpw
        pltpu.sync_copy(srcp.at[pl.ds(base, cpw)], src_v)
        pltpu.sync_copy(dstp.at[pl.ds(base, cpw)], dst_v)
        _fill(bufs[0], 0.0, chunk, w)
        for k in range(RPT // chunk):
            pltpu.sync_copy(bufs[0], acc.at[pl.ds(s * RPT + k * chunk, chunk)])
        plsc.subcore_barrier()

        for b in range(nbuf):
            pltpu.async_copy(t_hbm.at[src_v.at[b]], bufs[b], sems[b])

        def step(i, carry):
            j = nbuf * i
            for b in range(nbuf):
                jj = j + b
                pltpu.make_async_copy(t_hbm.at[src_v.at[jj]], bufs[b], sems[b]).wait()
                pltpu.sync_copy(bufs[b], acc.at[dst_v.at[jj]], add=True)
                pltpu.async_copy(t_hbm.at[src_v.at[jj + nbuf]], bufs[b], sems[b])
            return carry

        lax.fori_loop(0, cpw // nbuf - 1, step, 0)
        for b in range(nbuf):
            jj = cpw - nbuf + b
            pltpu.make_async_copy(t_hbm.at[src_v.at[jj]], bufs[b], sems[b]).wait()
            pltpu.sync_copy(bufs[b], acc.at[dst_v.at[jj]], add=True)
        if extra:
            @pl.when(wid < extra)
            def _():
                pltpu.sync_copy(srcp.at[pl.ds(NW * cpw + wid, 1)], ex_v)
                pltpu.sync_copy(dstp.at[pl.ds(NW * cpw + wid, 1)], exd_v)
                cp = pltpu.async_copy(t_hbm.at[ex_v.at[0]], bufs[0], sems[0])
                cp.wait()
                pltpu.sync_copy(bufs[0], acc.at[exd_v.at[0]], add=True)
        plsc.subcore_barrier()
        if w == 128:
            pltpu.sync_copy(acc.at[pl.ds(s * RPT, RPT)],
                            out.at[c, pl.ds(s * RPT, RPT)])
        else:
            pltpu.sync_copy(acc.at[pl.ds(s * RPT, RPT)],
                            out.at[pl.ds(s * RPT, RPT), pl.ds(c * w, w)])

    out_sh = ((NC, N_PAD, 128) if w == 128 else (N_PAD, 128))
    return pl.kernel(
        body,
        out_type=jax.ShapeDtypeStruct(out_sh, jnp.float32),
        mesh=_mesh(),
        compiler_params=pltpu.CompilerParams(use_tc_tiling_on_sc=False),
        scratch_types=(
            [pltpu.VMEM_SHARED((N_PAD, w), jnp.float32),
             pltpu.VMEM((cpw, chunk), jnp.int32),
             pltpu.VMEM((cpw, chunk), jnp.int32),
             pltpu.VMEM((1, chunk), jnp.int32),
             pltpu.VMEM((1, chunk), jnp.int32)]
            + [pltpu.VMEM((chunk, w), jnp.float32)] * nbuf
            + [pltpu.SemaphoreType.DMA] * nbuf
        ),
    )


def _indeg_kernel():
    chunk = 128
    nrows = E // chunk
    cpw = nrows // NW
    extra = nrows - NW * cpw

    def body(dstp, out, acc, dst_v, ex_v, ones_v, zero_v, sem):
        c = lax.axis_index("c")
        s = lax.axis_index("s")
        wid = c * NS + s
        base = wid * cpw
        pltpu.sync_copy(dstp.at[pl.ds(base, cpw)], dst_v)
        _fill(ones_v, 1.0, chunk, DEGW)
        _fill(zero_v, 0.0, chunk, DEGW)
        for k in range(RPT // chunk):
            pltpu.sync_copy(zero_v, acc.at[pl.ds(s * RPT + k * chunk, chunk)])
        plsc.subcore_barrier()

        def step(j, carry):
            pltpu.async_copy(ones_v, acc.at[dst_v.at[j]], sem, add=True)
            return carry

        lax.fori_loop(0, cpw, step, 0)

        def drain(j, carry):
            pltpu.make_async_copy(ones_v, acc.at[dst_v.at[j]], sem).wait()
            return carry

        lax.fori_loop(0, cpw, drain, 0)
        if extra:
            @pl.when(wid < extra)
            def _():
                pltpu.sync_copy(dstp.at[pl.ds(NW * cpw + wid, 1)], ex_v)
                pltpu.sync_copy(ones_v, acc.at[ex_v.at[0]], add=True)
        plsc.subcore_barrier()
        pltpu.sync_copy(acc.at[pl.ds(s * RPT, RPT)],
                        out.at[pl.ds(s * RPT, RPT), pl.ds(c * DEGW, DEGW)])

    return pl.kernel(
        body,
        out_type=jax.ShapeDtypeStruct((N_PAD, 2 * DEGW), jnp.float32),
        mesh=_mesh(),
        compiler_params=pltpu.CompilerParams(use_tc_tiling_on_sc=False),
        scratch_types=[
            pltpu.VMEM_SHARED((N_PAD, DEGW), jnp.float32),
            pltpu.VMEM((cpw, chunk), jnp.int32),
            pltpu.VMEM((1, chunk), jnp.int32),
            pltpu.VMEM((chunk, DEGW), jnp.float32),
            pltpu.VMEM((chunk, DEGW), jnp.float32),
            pltpu.SemaphoreType.DMA,
        ],
    )


RB = 2000
GRID = (N // RB,)


def _rspec(w):
    return pl.BlockSpec((RB, w), lambda i: (i, 0))


def _dspec():
    return pl.BlockSpec((RB, 2 * DEGW), lambda i: (i, 0))


def _pspec(w):
    return pl.BlockSpec((RB, 128), lambda i: (i, 0))


def _pspec3(w):
    return pl.BlockSpec((NC, RB, w), lambda i: (0, i, 0))


def _psum(p, w):
    return p[:, :w] + p[:, w:2 * w]


def _wspec(shape):
    return pl.BlockSpec(shape, lambda i: (0,) * len(shape))


def _scales(pdeg):
    indeg = pdeg[:, 0:1] + pdeg[:, DEGW:DEGW + 1]
    dinv = lax.rsqrt(indeg + 2.0)
    sc1 = dinv / (indeg + 1.0)
    return dinv, sc1


def _elu(v):
    return jnp.where(v > 0, v, jnp.exp(jnp.minimum(v, 0.0)) - 1.0)


def _sig(v):
    return 1.0 / (1.0 + jnp.exp(-v))


def _gate(z, oc):
    return _elu(z[:, :oc]) * _sig(z[:, oc:])


def _init_body(x_ref, w_ref, o_ref):
    o_ref[...] = jnp.dot(x_ref[...], w_ref[...], preferred_element_type=jnp.float32)


def _pre0_body(pdeg_ref, u_ref, o_ref):
    dinv, _ = _scales(pdeg_ref[...])
    o_ref[...] = dinv * u_ref[...]


def _stage0_body(pdeg_ref, p_ref, t_ref, out0_ref, t1_ref):
    dinv, sc1 = _scales(pdeg_ref[...])
    z = sc1 * (_psum(p_ref[...], 64) + 2.0 * t_ref[...])
    g = _gate(z, 32)
    out0_ref[...] = g
    t1_ref[...] = dinv * g


def _stage_mid(pdeg_ref, p_ref, t_ref, w_ref, o_refs, oc, skip_ref=None):
    dinv, sc1 = _scales(pdeg_ref[...])
    p = p_ref[...]
    psum = (p[0] + p[1]) if p.ndim == 3 else _psum(p, t_ref.shape[-1])
    y = sc1 * (psum + 2.0 * t_ref[...])
    h = jnp.dot(y, w_ref[...], preferred_element_type=jnp.float32)
    g = _gate(h, oc)
    if skip_ref is not None:
        g = g + skip_ref[...]
    if len(o_refs) == 2:
        o_refs[0][...] = g
        o_refs[1][...] = dinv * g
    else:
        o_refs[0][...] = dinv * g


def _stage1_body(pdeg_ref, p_ref, t_ref, w_ref, out_ref, tn_ref):
    _stage_mid(pdeg_ref, p_ref, t_ref, w_ref, (out_ref, tn_ref), 64)


def _stage2_body(pdeg_ref, p_ref, t_ref, w_ref, tn_ref):
    _stage_mid(pdeg_ref, p_ref, t_ref, w_ref, (tn_ref,), 128)


def _stage3_body(pdeg_ref, p_ref, t_ref, w_ref, skip_ref, tn_ref):
    _stage_mid(pdeg_ref, p_ref, t_ref, w_ref, (tn_ref,), 64, skip_ref)


def _stage4_body(pdeg_ref, p_ref, t_ref, w_ref, skip_ref, w5_ref, t5_ref):
    dinv, sc1 = _scales(pdeg_ref[...])
    y = sc1 * (_psum(p_ref[...], 64) + 2.0 * t_ref[...])
    h = jnp.dot(y, w_ref[...], preferred_element_type=jnp.float32)
    g = _gate(h, 32) + skip_ref[...]
    u5 = jnp.dot(g, w5_ref[...], preferred_element_type=jnp.float32)
    t5_ref[...] = dinv * u5


def _stage5_body(pdeg_ref, p_ref, t_ref, o_ref):
    _, sc1 = _scales(pdeg_ref[...])
    z = sc1 * (_psum(p_ref[...], DEGW) + 2.0 * t_ref[...])
    o_ref[...] = _sig(z[:, :3])


def _f32(shape):
    return jax.ShapeDtypeStruct(shape, jnp.float32)


def kernel(x, edge_index, W0a, W0b, W1a, W1b, W2a, W2b, W3a, W3b, W4a, W4b, W5):
    src = edge_index[0].astype(jnp.int32)
    dst = edge_index[1].astype(jnp.int32)
    srcp = src.reshape(-1, 128)
    dstp = dst.reshape(-1, 128)
    srcp40 = src.reshape(-1, 40)
    dstp40 = dst.reshape(-1, 40)

    W0 = jnp.concatenate([W0a, W0b], axis=1)
    W1 = jnp.concatenate([W1a, W1b], axis=1)
    W2 = jnp.concatenate([W2a, W2b], axis=1)
    W3 = jnp.concatenate([W3a, W3b], axis=1)
    W4 = jnp.concatenate([W4a, W4b], axis=1)
    W5p = jnp.pad(W5, ((0, 0), (0, DEGW - 3)))

    pdeg = _indeg_kernel()(dstp)

    u0 = pl.pallas_call(
        _init_body, grid=GRID,
        in_specs=[_rspec(128), _wspec((128, 64))],
        out_specs=_rspec(64), out_shape=_f32((N, 64)),
    )(x, W0)

    t0 = pl.pallas_call(
        _pre0_body, grid=GRID,
        in_specs=[_dspec(), _rspec(64)],
        out_specs=_rspec(64), out_shape=_f32((N, 64)),
    )(pdeg, u0)

    p0 = _spmm(64)(t0, srcp, dstp)
    out0, t1 = pl.pallas_call(
        _stage0_body, grid=GRID,
        in_specs=[_dspec(), _pspec(64), _rspec(64)],
        out_specs=(_rspec(32), _rspec(32)),
        out_shape=(_f32((N, 32)), _f32((N, 32))),
    )(pdeg, p0, t0)

    p1 = _spmm(32)(t1, srcp, dstp)
    out1, t2 = pl.pallas_call(
        _stage1_body, grid=GRID,
        in_specs=[_dspec(), _pspec(32), _rspec(32), _wspec((32, 128))],
        out_specs=(_rspec(64), _rspec(64)),
        out_shape=(_f32((N, 64)), _f32((N, 64))),
    )(pdeg, p1, t1, W1)

    p2 = _spmm(64)(t2, srcp, dstp)
    (t3,) = pl.pallas_call(
        _stage2_body, grid=GRID,
        in_specs=[_dspec(), _pspec(64), _rspec(64), _wspec((64, 256))],
        out_specs=(_rspec(128),),
        out_shape=(_f32((N, 128)),),
    )(pdeg, p2, t2, W2)

    p3 = _spmm(128)(t3, srcp40, dstp40)
    (t4,) = pl.pallas_call(
        _stage3_body, grid=GRID,
        in_specs=[_dspec(), _pspec3(128), _rspec(128), _wspec((128, 128)),
                  _rspec(64)],
        out_specs=(_rspec(64),),
        out_shape=(_f32((N, 64)),),
    )(pdeg, p3, t3, W3, out1)

    p4 = _spmm(64)(t4, srcp, dstp)
    (t5,) = pl.pallas_call(
        _stage4_body, grid=GRID,
        in_specs=[_dspec(), _pspec(64), _rspec(64), _wspec((64, 64)),
                  _rspec(32), _wspec((32, DEGW))],
        out_specs=(_rspec(DEGW),),
        out_shape=(_f32((N, DEGW)),),
    )(pdeg, p4, t4, W4, out0, W5p)

    p5 = _spmm(DEGW)(t5, srcp, dstp)
    return pl.pallas_call(
        _stage5_body, grid=GRID,
        in_specs=[_dspec(), _pspec(DEGW), _rspec(DEGW)],
        out_specs=_rspec(3), out_shape=_f32((N, 3)),
    )(pdeg, p5, t5)


# speedup vs baseline: 1.1199x; 1.0100x over previous
import functools

import jax
import jax.numpy as jnp
from jax import lax
from jax.experimental import pallas as pl
from jax.experimental.pallas import tpu as pltpu
from jax.experimental.pallas import tpu_sc as plsc

N = 10000
E = 320000
NC, NS, LANES = 2, 16, 16
NW = NC * NS


def _nbuf(w):
    if w == 128:
        return 5
    return 13 if w <= 32 else 6


def _chunk(w):
    return 40 if w == 128 else 128
N_PAD = 10240
RPT = N_PAD // NS
DEGW = 16

@functools.cache
def _mesh():
    return plsc.VectorSubcoreMesh(core_axis_name="c", subcore_axis_name="s")


def _fill(ref, val, rows, w):
    def body(r, carry):
        for cc in range(w // LANES):
            ref[r, pl.ds(cc * LANES, LANES)] = jnp.full((LANES,), val, jnp.float32)
        return carry
    lax.fori_loop(0, rows, body, 0)


@functools.cache
def _spmm(w):
    chunk = _chunk(w)
    nbuf = _nbuf(w)
    nrows = E // chunk
    cpw = nrows // NW
    extra = nrows - NW * cpw
    assert cpw % nbuf == 0

    def body(t_hbm, srcp, dstp, out, acc, src_v, dst_v, ex_v, exd_v, *bufsems):
        bufs, sems = bufsems[:nbuf], bufsems[nbuf:]
        c = lax.axis_index("c")
        s = lax.axis_index("s")
        wid = c * NS + s
        base = wid * cpw
        pltpu.sync_copy(srcp.at[pl.ds(base, cpw)], src_v)
        pltpu.sync_copy(dstp.at[pl.ds(base, cpw)], dst_v)
        _fill(bufs[0], 0.0, chunk, w)
        for k in range(RPT // chunk):
            pltpu.sync_copy(bufs[0], acc.at[pl.ds(s * RPT + k * chunk, chunk)])
        plsc.subcore_barrier()

        for b in range(nbuf):
            pltpu.async_copy(t_hbm.at[src_v.at[b]], bufs[b], sems[b])

        def step(i, carry):
            j = nbuf * i
            for b in range(nbuf):
                jj = j + b
                pltpu.make_async_copy(t_hbm.at[src_v.at[jj]], bufs[b], sems[b]).wait()
                pltpu.sync_copy(bufs[b], acc.at[dst_v.at[jj]], add=True)
                pltpu.async_copy(t_hbm.at[src_v.at[jj + nbuf]], bufs[b], sems[b])
            return carry

        lax.fori_loop(0, cpw // nbuf - 1, step, 0)
        for b in range(nbuf):
            jj = cpw - nbuf + b
            pltpu.make_async_copy(t_hbm.at[src_v.at[jj]], bufs[b], sems[b]).wait()
            pltpu.sync_copy(bufs[b], acc.at[dst_v.at[jj]], add=True)
        if extra:
            @pl.when(wid < extra)
            def _():
                pltpu.sync_copy(srcp.at[pl.ds(NW * cpw + wid, 1)], ex_v)
                pltpu.sync_copy(dstp.at[pl.ds(NW * cpw + wid, 1)], exd_v)
                cp = pltpu.async_copy(t_hbm.at[ex_v.at[0]], bufs[0], sems[0])
                cp.wait()
                pltpu.sync_copy(bufs[0], acc.at[exd_v.at[0]], add=True)
        plsc.subcore_barrier()
        if w == 128:
            pltpu.sync_copy(acc.at[pl.ds(s * RPT, RPT)],
                            out.at[c, pl.ds(s * RPT, RPT)])
        else:
            pltpu.sync_copy(acc.at[pl.ds(s * RPT, RPT)],
                            out.at[pl.ds(s * RPT, RPT), pl.ds(c * w, w)])

    out_sh = ((NC, N_PAD, 128) if w == 128 else (N_PAD, 128))
    return pl.kernel(
        body,
        out_type=jax.ShapeDtypeStruct(out_sh, jnp.float32),
        mesh=_mesh(),
        compiler_params=pltpu.CompilerParams(use_tc_tiling_on_sc=False),
        scratch_types=(
            [pltpu.VMEM_SHARED((N_PAD, w), jnp.float32),
             pltpu.VMEM((cpw, chunk), jnp.int32),
             pltpu.VMEM((cpw, chunk), jnp.int32),
             pltpu.VMEM((1, chunk), jnp.int32),
             pltpu.VMEM((1, chunk), jnp.int32)]
            + [pltpu.VMEM((chunk, w), jnp.float32)] * nbuf
            + [pltpu.SemaphoreType.DMA] * nbuf
        ),
    )


def _indeg_kernel():
    chunk = 128
    nrows = E // chunk
    cpw = nrows // NW
    extra = nrows - NW * cpw

    def body(dstp, out, acc, dst_v, ex_v, ones_v, zero_v, sem):
        c = lax.axis_index("c")
        s = lax.axis_index("s")
        wid = c * NS + s
        base = wid * cpw
        pltpu.sync_copy(dstp.at[pl.ds(base, cpw)], dst_v)
        _fill(ones_v, 1.0, chunk, DEGW)
        _fill(zero_v, 0.0, chunk, DEGW)
        for k in range(RPT // chunk):
            pltpu.sync_copy(zero_v, acc.at[pl.ds(s * RPT + k * chunk, chunk)])
        plsc.subcore_barrier()

        def step(j, carry):
            pltpu.async_copy(ones_v, acc.at[dst_v.at[j]], sem, add=True)
            return carry

        lax.fori_loop(0, cpw, step, 0)

        def drain(j, carry):
            pltpu.make_async_copy(ones_v, acc.at[dst_v.at[j]], sem).wait()
            return carry

        lax.fori_loop(0, cpw, drain, 0)
        if extra:
            @pl.when(wid < extra)
            def _():
                pltpu.sync_copy(dstp.at[pl.ds(NW * cpw + wid, 1)], ex_v)
                pltpu.sync_copy(ones_v, acc.at[ex_v.at[0]], add=True)
        plsc.subcore_barrier()
        pltpu.sync_copy(acc.at[pl.ds(s * RPT, RPT)],
                        out.at[pl.ds(s * RPT, RPT), pl.ds(c * DEGW, DEGW)])

    return pl.kernel(
        body,
        out_type=jax.ShapeDtypeStruct((N_PAD, 2 * DEGW), jnp.float32),
        mesh=_mesh(),
        compiler_params=pltpu.CompilerParams(use_tc_tiling_on_sc=False),
        scratch_types=[
            pltpu.VMEM_SHARED((N_PAD, DEGW), jnp.float32),
            pltpu.VMEM((cpw, chunk), jnp.int32),
            pltpu.VMEM((1, chunk), jnp.int32),
            pltpu.VMEM((chunk, DEGW), jnp.float32),
            pltpu.VMEM((chunk, DEGW), jnp.float32),
            pltpu.SemaphoreType.DMA,
        ],
    )


RB = 5000
GRID = (N // RB,)


def _rspec(w):
    return pl.BlockSpec((RB, w), lambda i: (i, 0))


def _dspec():
    return pl.BlockSpec((RB, 2 * DEGW), lambda i: (i, 0))


def _pspec(w):
    return pl.BlockSpec((RB, 128), lambda i: (i, 0))


def _pspec3(w):
    return pl.BlockSpec((NC, RB, w), lambda i: (0, i, 0))


def _psum(p, w):
    return p[:, :w] + p[:, w:2 * w]


def _wspec(shape):
    return pl.BlockSpec(shape, lambda i: (0,) * len(shape))


def _scales(pdeg):
    indeg = pdeg[:, 0:1] + pdeg[:, DEGW:DEGW + 1]
    dinv = lax.rsqrt(indeg + 2.0)
    sc1 = dinv / (indeg + 1.0)
    return dinv, sc1


def _elu(v):
    return jnp.where(v > 0, v, jnp.exp(jnp.minimum(v, 0.0)) - 1.0)


def _sig(v):
    return 1.0 / (1.0 + jnp.exp(-v))


def _gate(z, oc):
    return _elu(z[:, :oc]) * _sig(z[:, oc:])


def _init_body(x_ref, w_ref, o_ref):
    o_ref[...] = jnp.dot(x_ref[...], w_ref[...], preferred_element_type=jnp.float32)


def _pre0_body(pdeg_ref, u_ref, o_ref):
    dinv, _ = _scales(pdeg_ref[...])
    o_ref[...] = dinv * u_ref[...]


def _stage0_body(pdeg_ref, p_ref, t_ref, out0_ref, t1_ref):
    dinv, sc1 = _scales(pdeg_ref[...])
    z = sc1 * (_psum(p_ref[...], 64) + 2.0 * t_ref[...])
    g = _gate(z, 32)
    out0_ref[...] = g
    t1_ref[...] = dinv * g


def _stage_mid(pdeg_ref, p_ref, t_ref, w_ref, o_refs, oc, skip_ref=None):
    dinv, sc1 = _scales(pdeg_ref[...])
    p = p_ref[...]
    psum = (p[0] + p[1]) if p.ndim == 3 else _psum(p, t_ref.shape[-1])
    y = sc1 * (psum + 2.0 * t_ref[...])
    h = jnp.dot(y, w_ref[...], preferred_element_type=jnp.float32)
    g = _gate(h, oc)
    if skip_ref is not None:
        g = g + skip_ref[...]
    if len(o_refs) == 2:
        o_refs[0][...] = g
        o_refs[1][...] = dinv * g
    else:
        o_refs[0][...] = dinv * g


def _stage1_body(pdeg_ref, p_ref, t_ref, w_ref, out_ref, tn_ref):
    _stage_mid(pdeg_ref, p_ref, t_ref, w_ref, (out_ref, tn_ref), 64)


def _stage2_body(pdeg_ref, p_ref, t_ref, w_ref, tn_ref):
    _stage_mid(pdeg_ref, p_ref, t_ref, w_ref, (tn_ref,), 128)


def _stage3_body(pdeg_ref, p_ref, t_ref, w_ref, skip_ref, tn_ref):
    _stage_mid(pdeg_ref, p_ref, t_ref, w_ref, (tn_ref,), 64, skip_ref)


def _stage4_body(pdeg_ref, p_ref, t_ref, w_ref, skip_ref, w5_ref, t5_ref):
    dinv, sc1 = _scales(pdeg_ref[...])
    y = sc1 * (_psum(p_ref[...], 64) + 2.0 * t_ref[...])
    h = jnp.dot(y, w_ref[...], preferred_element_type=jnp.float32)
    g = _gate(h, 32) + skip_ref[...]
    u5 = jnp.dot(g, w5_ref[...], preferred_element_type=jnp.float32)
    t5_ref[...] = dinv * u5


def _stage5_body(pdeg_ref, p_ref, t_ref, o_ref):
    _, sc1 = _scales(pdeg_ref[...])
    z = sc1 * (_psum(p_ref[...], DEGW) + 2.0 * t_ref[...])
    o_ref[...] = _sig(z[:, :3])


def _f32(shape):
    return jax.ShapeDtypeStruct(shape, jnp.float32)


def kernel(x, edge_index, W0a, W0b, W1a, W1b, W2a, W2b, W3a, W3b, W4a, W4b, W5):
    src = edge_index[0].astype(jnp.int32)
    dst = edge_index[1].astype(jnp.int32)
    srcp = src.reshape(-1, 128)
    dstp = dst.reshape(-1, 128)
    srcp40 = src.reshape(-1, 40)
    dstp40 = dst.reshape(-1, 40)

    W0 = jnp.concatenate([W0a, W0b], axis=1)
    W1 = jnp.concatenate([W1a, W1b], axis=1)
    W2 = jnp.concatenate([W2a, W2b], axis=1)
    W3 = jnp.concatenate([W3a, W3b], axis=1)
    W4 = jnp.concatenate([W4a, W4b], axis=1)
    W5p = jnp.pad(W5, ((0, 0), (0, DEGW - 3)))

    pdeg = _indeg_kernel()(dstp)

    u0 = pl.pallas_call(
        _init_body, grid=GRID,
        in_specs=[_rspec(128), _wspec((128, 64))],
        out_specs=_rspec(64), out_shape=_f32((N, 64)),
    )(x, W0)

    t0 = pl.pallas_call(
        _pre0_body, grid=GRID,
        in_specs=[_dspec(), _rspec(64)],
        out_specs=_rspec(64), out_shape=_f32((N, 64)),
    )(pdeg, u0)

    p0 = _spmm(64)(t0, srcp, dstp)
    out0, t1 = pl.pallas_call(
        _stage0_body, grid=GRID,
        in_specs=[_dspec(), _pspec(64), _rspec(64)],
        out_specs=(_rspec(32), _rspec(32)),
        out_shape=(_f32((N, 32)), _f32((N, 32))),
    )(pdeg, p0, t0)

    p1 = _spmm(32)(t1, srcp, dstp)
    out1, t2 = pl.pallas_call(
        _stage1_body, grid=GRID,
        in_specs=[_dspec(), _pspec(32), _rspec(32), _wspec((32, 128))],
        out_specs=(_rspec(64), _rspec(64)),
        out_shape=(_f32((N, 64)), _f32((N, 64))),
    )(pdeg, p1, t1, W1)

    p2 = _spmm(64)(t2, srcp, dstp)
    (t3,) = pl.pallas_call(
        _stage2_body, grid=GRID,
        in_specs=[_dspec(), _pspec(64), _rspec(64), _wspec((64, 256))],
        out_specs=(_rspec(128),),
        out_shape=(_f32((N, 128)),),
    )(pdeg, p2, t2, W2)

    p3 = _spmm(128)(t3, srcp40, dstp40)
    (t4,) = pl.pallas_call(
        _stage3_body, grid=GRID,
        in_specs=[_dspec(), _pspec3(128), _rspec(128), _wspec((128, 128)),
                  _rspec(64)],
        out_specs=(_rspec(64),),
        out_shape=(_f32((N, 64)),),
    )(pdeg, p3, t3, W3, out1)

    p4 = _spmm(64)(t4, srcp, dstp)
    (t5,) = pl.pallas_call(
        _stage4_body, grid=GRID,
        in_specs=[_dspec(), _pspec(64), _rspec(64), _wspec((64, 64)),
                  _rspec(32), _wspec((32, DEGW))],
        out_specs=(_rspec(DEGW),),
        out_shape=(_f32((N, DEGW)),),
    )(pdeg, p4, t4, W4, out0, W5p)

    p5 = _spmm(DEGW)(t5, srcp, dstp)
    return pl.pallas_call(
        _stage5_body, grid=GRID,
        in_specs=[_dspec(), _pspec(DEGW), _rspec(DEGW)],
        out_specs=_rspec(3), out_shape=_f32((N, 3)),
    )(pdeg, p5, t5)
